# baseline (speedup 1.0000x reference)
import functools

import jax
import jax.numpy as jnp
from jax import lax
from jax.experimental import pallas as pl
from jax.experimental.pallas import tpu as pltpu
from jax.experimental.pallas import tpu_sc as plsc

_N = 8192
_H = 1024
_K = 819
_NS = 16
_PER = _N // _NS
_NV = _PER // 16
_BROWS = 256
_ZROWS = 32
_GROUPS = _BROWS // 16
_RPAD = 48


def _avg_floor(a, b):
    return (a & b) + ((a ^ b) >> 1)


def _orderable(x_f32):
    i = lax.bitcast_convert_type(x_f32, jnp.int32)
    return jnp.where(i < 0, i ^ jnp.int32(0x7FFFFFFF), i)


def _sc_body(keys_hbm, values_hbm, scores_hbm, ts_hbm, out_hbm,
             sbuf, tbuf, us, ut, zbuf, idxk, idxv, idxze, idxzv, stagek, stagev,
             stagek2, stagev2, smem2, sem_z, sem_g, sem_g2, sem_s, sem_s2):
    c = lax.axis_index("c")
    s = lax.axis_index("s")
    row0 = s * _PER
    brow0 = row0 + c * _BROWS
    lane = lax.broadcasted_iota(jnp.int32, (16,), 0)
    zero16 = jnp.zeros((16,), jnp.int32)

    def zinit(i, _):
        r = i >> 6
        w = i & 63
        zbuf[r, pl.ds(w * 16, 16)] = jnp.zeros((16,), jnp.float32)
        return 0

    lax.fori_loop(0, 16 * _H // 16, zinit, 0)

    pltpu.sync_copy(scores_hbm.at[pl.ds(row0, _PER)], sbuf)
    pltpu.sync_copy(ts_hbm.at[pl.ds(row0, _PER)], tbuf)

    def mk(i, _):
        us[pl.ds(i * 16, 16)] = _orderable(sbuf[pl.ds(i * 16, 16)])
        ut[pl.ds(i * 16, 16)] = _orderable(tbuf[pl.ds(i * 16, 16)])
        return 0

    lax.fori_loop(0, _NV, mk, 0)

    def _lane0(v):
        return lax.squeeze(lax.slice(v, (0,), (1,)), (0,))

    @pl.when(s == 0)
    def _():
        def zc(r, _):
            smem2[r] = 0
            return 0

        lax.fori_loop(0, 2 * _RPAD, zc, 0)

    plsc.subcore_barrier()

    def _global_pair(r, cs, ct):
        packed = _lane0(cs) + (_lane0(ct) << 16)
        plsc.fetch_and_add(smem2.at[r], packed, subcore_id=0)
        plsc.subcore_barrier()
        tot = plsc.fetch_and_add(smem2.at[r], 0, subcore_id=0)
        return tot & 0xFFFF, tot >> 16

    def vround(r, carry):
        lo_s, hi_s, lo_t, hi_t = carry
        mid_s = _avg_floor(lo_s, hi_s) + ((lo_s ^ hi_s) & 1)
        mid_t = _avg_floor(lo_t, hi_t) + ((lo_t ^ hi_t) & 1)

        def cnt(i, acc):
            a, b = acc
            a = a + plsc.all_reduce_population_count(us[pl.ds(i * 16, 16)] >= mid_s)
            b = b + plsc.all_reduce_population_count(ut[pl.ds(i * 16, 16)] >= mid_t)
            return (a, b)

        cs, ct = lax.fori_loop(0, _NV, cnt, (zero16, zero16))
        cs_tot, ct_tot = _global_pair(r, cs, ct)

        go_s = lo_s < hi_s
        take_s = cs_tot >= _K
        lo_s = jnp.where(go_s & take_s, mid_s, lo_s)
        hi_s = jnp.where(go_s & (~take_s), mid_s - 1, hi_s)
        go_t = lo_t < hi_t
        take_t = ct_tot >= _K
        lo_t = jnp.where(go_t & take_t, mid_t, lo_t)
        hi_t = jnp.where(go_t & (~take_t), mid_t - 1, hi_t)
        return (lo_s, hi_s, lo_t, hi_t)

    int_min = jnp.int32(-2147483648)
    int_max = jnp.int32(2147483647)
    lo_s, _, lo_t, _ = lax.fori_loop(
        0, 32, vround, (int_min, int_max, int_min, int_max))
    Ts, Tt = lo_s, lo_t

    def cntgt(i, acc):
        a, b = acc
        a = a + plsc.all_reduce_population_count(us[pl.ds(i * 16, 16)] > Ts)
        b = b + plsc.all_reduce_population_count(ut[pl.ds(i * 16, 16)] > Tt)
        return (a, b)

    gs, gt = lax.fori_loop(0, _NV, cntgt, (zero16, zero16))
    gs_tot, gt_tot = _global_pair(32, gs, gt)
    ne_s = _K - gs_tot
    ne_t = _K - gt_tot

    def iround(r, carry):
        lo2s, hi2s, lo2t, hi2t = carry
        mid_s = _avg_floor(lo2s, hi2s)
        mid_t = _avg_floor(lo2t, hi2t)

        def cnt(i, acc):
            a, b = acc
            gidx = row0 + i * 16 + lane
            a = a + plsc.all_reduce_population_count(
                (us[pl.ds(i * 16, 16)] == Ts) & (gidx <= mid_s))
            b = b + plsc.all_reduce_population_count(
                (ut[pl.ds(i * 16, 16)] == Tt) & (gidx <= mid_t))
            return (a, b)

        cs, ct = lax.fori_loop(0, _NV, cnt, (zero16, zero16))
        cs_tot, ct_tot = _global_pair(33 + r, cs, ct)

        go_s = lo2s < hi2s
        take_s = cs_tot >= ne_s
        hi2s = jnp.where(go_s & take_s, mid_s, hi2s)
        lo2s = jnp.where(go_s & (~take_s), mid_s + 1, lo2s)
        go_t = lo2t < hi2t
        take_t = ct_tot >= ne_t
        hi2t = jnp.where(go_t & take_t, mid_t, hi2t)
        lo2t = jnp.where(go_t & (~take_t), mid_t + 1, lo2t)
        return (lo2s, hi2s, lo2t, hi2t)

    Ps, _, Pt, _ = lax.fori_loop(
        0, 14, iround,
        (jnp.int32(0), jnp.int32(_N - 1), jnp.int32(0), jnp.int32(_N - 1)))

    def comp(i, carry):
        n, m = carry
        iv = c * (_BROWS // 16) + i
        uv = us[pl.ds(iv * 16, 16)]
        tv = ut[pl.ds(iv * 16, 16)]
        gidx = row0 + iv * 16 + lane
        keep = ((uv > Ts) | ((uv == Ts) & (gidx <= Ps)) |
                (tv > Tt) | ((tv == Tt) & (gidx <= Pt)))
        ki = keep.astype(jnp.int32)
        incl = plsc.cumsum(ki)
        pos = n + incl - ki
        plsc.store_scatter(idxk, [pos >> 4, pos & 15], gidx, mask=keep)
        plsc.store_scatter(idxv, [pos >> 4, pos & 15], _N + gidx, mask=keep)
        ev = ~keep
        ei = ev.astype(jnp.int32)
        incle = plsc.cumsum(ei)
        pose = m + incle - ei
        plsc.store_scatter(idxze, [pose >> 4, pose & 15], gidx, mask=ev)
        plsc.store_scatter(idxzv, [pose >> 4, pose & 15], _N + gidx, mask=ev)
        return (n + plsc.all_reduce_population_count(keep),
                m + plsc.all_reduce_population_count(ev))

    n, m = lax.fori_loop(0, _GROUPS, comp, (zero16, zero16))
    ngroups = (_lane0(n) + 15) >> 4
    ngroups_ev = (_lane0(m) + 15) >> 4

    fk = _lane0(idxk[0, :])
    fe = _lane0(idxze[0, :])

    def padfix(g, _):
        mm = (g * 16 + lane) >= n
        idxk[g, :] = jnp.where(mm, fk, idxk[g, :])
        idxv[g, :] = jnp.where(mm, _N + fk, idxv[g, :])
        me = (g * 16 + lane) >= m
        idxze[g, :] = jnp.where(me, fe, idxze[g, :])
        idxzv[g, :] = jnp.where(me, _N + fe, idxzv[g, :])
        return 0

    lax.fori_loop(0, _GROUPS, padfix, 0)

    for g in range(_GROUPS):
        @pl.when(g < ngroups_ev)
        def _(g=g):
            pltpu.make_async_copy(zbuf, out_hbm.at[idxze.at[g]], sem_z).start()
            pltpu.make_async_copy(zbuf, out_hbm.at[idxzv.at[g]], sem_z).start()

    stk = (stagek, stagek2)
    stv = (stagev, stagev2)
    sems = (sem_g, sem_g2)

    def _gather(g, b):
        kcp = pltpu.make_async_copy(keys_hbm.at[idxk.at[g]], stk[b], sems[b])
        vcp = pltpu.make_async_copy(values_hbm.at[idxk.at[g]], stv[b], sems[b])
        kcp.start()
        vcp.start()

    def _drain(g, b):
        pltpu.make_async_copy(keys_hbm.at[idxk.at[g]], stk[b], sems[b]).wait()
        pltpu.make_async_copy(values_hbm.at[idxk.at[g]], stv[b], sems[b]).wait()

    @pl.when(0 < ngroups)
    def _():
        _gather(0, 0)

    ssems = (sem_s, sem_s2)

    def _scatter(g, b):
        kcp = pltpu.make_async_copy(stk[b], out_hbm.at[idxk.at[g]], ssems[b])
        vcp = pltpu.make_async_copy(stv[b], out_hbm.at[idxv.at[g]], ssems[b])
        kcp.start()
        vcp.start()

    def _sdrain(g, b):
        pltpu.make_async_copy(stk[b], out_hbm.at[idxk.at[g]], ssems[b]).wait()
        pltpu.make_async_copy(stv[b], out_hbm.at[idxv.at[g]], ssems[b]).wait()

    for g in range(_GROUPS):
        b = g & 1

        @pl.when(g < ngroups)
        def _(g=g, b=b):
            _drain(g, b)
            if g + 1 < _GROUPS:
                @pl.when(g + 1 < ngroups)
                def _():
                    if g >= 1:
                        _sdrain(g - 1, 1 - b)
                    _gather(g + 1, 1 - b)
            _scatter(g, b)

    for g in range(_GROUPS):
        b = g & 1

        @pl.when((g < ngroups) & (g + 2 >= ngroups))
        def _(g=g, b=b):
            _sdrain(g, b)

    for g in range(_GROUPS):
        @pl.when(g < ngroups_ev)
        def _(g=g):
            pltpu.make_async_copy(zbuf, out_hbm.at[idxze.at[g]], sem_z).wait()
            pltpu.make_async_copy(zbuf, out_hbm.at[idxzv.at[g]], sem_z).wait()


@jax.jit
def kernel(keys, values, attention_scores, timestamps):
    mesh = plsc.VectorSubcoreMesh(
        core_axis_name="c", subcore_axis_name="s", num_cores=2)
    run = functools.partial(
        pl.kernel,
        out_type=jax.ShapeDtypeStruct((2 * _N, _H), jnp.float32),
        mesh=mesh,
        compiler_params=pltpu.CompilerParams(needs_layout_passes=False),
        scratch_types=[
            pltpu.VMEM((_PER,), jnp.float32),
            pltpu.VMEM((_PER,), jnp.float32),
            pltpu.VMEM((_PER,), jnp.int32),
            pltpu.VMEM((_PER,), jnp.int32),
            pltpu.VMEM((16, _H), jnp.float32),
            pltpu.VMEM((_GROUPS, 16), jnp.int32),
            pltpu.VMEM((_GROUPS, 16), jnp.int32),
            pltpu.VMEM((_GROUPS, 16), jnp.int32),
            pltpu.VMEM((_GROUPS, 16), jnp.int32),
            pltpu.VMEM((16, _H), jnp.float32),
            pltpu.VMEM((16, _H), jnp.float32),
            pltpu.VMEM((16, _H), jnp.float32),
            pltpu.VMEM((16, _H), jnp.float32),
            pltpu.SMEM((2 * _RPAD,), jnp.int32),
            pltpu.SemaphoreType.DMA,
            pltpu.SemaphoreType.DMA,
            pltpu.SemaphoreType.DMA,
            pltpu.SemaphoreType.DMA,
            pltpu.SemaphoreType.DMA,
        ],
    )(_sc_body)
    flat = run(keys, values, attention_scores, timestamps)
    return flat.reshape(2, _N, _H)


# speedup vs baseline: 1.1363x; 1.1363x over previous
import functools

import jax
import jax.numpy as jnp
from jax import lax
from jax.experimental import pallas as pl
from jax.experimental.pallas import tpu as pltpu
from jax.experimental.pallas import tpu_sc as plsc

_N = 8192
_H = 1024
_K = 819
_NS = 16
_PER = _N // _NS
_NV = _PER // 16
_BROWS = 256
_ZROWS = 32
_GROUPS = _BROWS // 16
_RPAD = 48


def _avg_floor(a, b):
    return (a & b) + ((a ^ b) >> 1)


def _orderable(x_f32):
    i = lax.bitcast_convert_type(x_f32, jnp.int32)
    return jnp.where(i < 0, i ^ jnp.int32(0x7FFFFFFF), i)


def _sc_body(keys_hbm, values_hbm, scores_hbm, ts_hbm, out_hbm,
             sbuf, tbuf, us, ut, zbuf, idxk, idxv, stagek, stagev,
             stagek2, stagev2, smem2, sem_z, sem_g, sem_g2, sem_s, sem_s2):
    c = lax.axis_index("c")
    s = lax.axis_index("s")
    row0 = s * _PER
    brow0 = row0 + c * _BROWS
    lane = lax.broadcasted_iota(jnp.int32, (16,), 0)
    zero16 = jnp.zeros((16,), jnp.int32)

    def zinit(i, _):
        r = i >> 6
        w = i & 63
        zbuf[r, pl.ds(w * 16, 16)] = jnp.zeros((16,), jnp.float32)
        return 0

    lax.fori_loop(0, _ZROWS * _H // 16, zinit, 0)

    zcopies = []
    for j in range(_BROWS // _ZROWS):
        for plane in range(2):
            dst = out_hbm.at[pl.ds(plane * _N + brow0 + j * _ZROWS, _ZROWS), :]
            cp = pltpu.make_async_copy(zbuf, dst, sem_z)
            cp.start()
            zcopies.append(cp)

    pltpu.sync_copy(scores_hbm.at[pl.ds(row0, _PER)], sbuf)
    pltpu.sync_copy(ts_hbm.at[pl.ds(row0, _PER)], tbuf)

    def mk(i, _):
        us[pl.ds(i * 16, 16)] = _orderable(sbuf[pl.ds(i * 16, 16)])
        ut[pl.ds(i * 16, 16)] = _orderable(tbuf[pl.ds(i * 16, 16)])
        return 0

    lax.fori_loop(0, _NV, mk, 0)

    def _lane0(v):
        return lax.squeeze(lax.slice(v, (0,), (1,)), (0,))

    @pl.when(s == 0)
    def _():
        def zc(r, _):
            smem2[r] = 0
            return 0

        lax.fori_loop(0, 2 * _RPAD, zc, 0)

    plsc.subcore_barrier()

    def _global_pair(r, cs, ct):
        packed = _lane0(cs) + (_lane0(ct) << 16)
        plsc.fetch_and_add(smem2.at[r], packed, subcore_id=0)
        plsc.subcore_barrier()
        tot = plsc.fetch_and_add(smem2.at[r], 0, subcore_id=0)
        return tot & 0xFFFF, tot >> 16

    def vround(r, carry):
        lo_s, hi_s, lo_t, hi_t = carry
        mid_s = _avg_floor(lo_s, hi_s) + ((lo_s ^ hi_s) & 1)
        mid_t = _avg_floor(lo_t, hi_t) + ((lo_t ^ hi_t) & 1)

        def cnt(i, acc):
            a, b = acc
            a = a + plsc.all_reduce_population_count(us[pl.ds(i * 16, 16)] >= mid_s)
            b = b + plsc.all_reduce_population_count(ut[pl.ds(i * 16, 16)] >= mid_t)
            return (a, b)

        cs, ct = lax.fori_loop(0, _NV, cnt, (zero16, zero16))
        cs_tot, ct_tot = _global_pair(r, cs, ct)

        go_s = lo_s < hi_s
        take_s = cs_tot >= _K
        lo_s = jnp.where(go_s & take_s, mid_s, lo_s)
        hi_s = jnp.where(go_s & (~take_s), mid_s - 1, hi_s)
        go_t = lo_t < hi_t
        take_t = ct_tot >= _K
        lo_t = jnp.where(go_t & take_t, mid_t, lo_t)
        hi_t = jnp.where(go_t & (~take_t), mid_t - 1, hi_t)
        return (lo_s, hi_s, lo_t, hi_t)

    int_min = jnp.int32(-2147483648)
    int_max = jnp.int32(2147483647)
    lo_s, _, lo_t, _ = lax.fori_loop(
        0, 32, vround, (int_min, int_max, int_min, int_max))
    Ts, Tt = lo_s, lo_t

    def cntgt(i, acc):
        a, b = acc
        a = a + plsc.all_reduce_population_count(us[pl.ds(i * 16, 16)] > Ts)
        b = b + plsc.all_reduce_population_count(ut[pl.ds(i * 16, 16)] > Tt)
        return (a, b)

    gs, gt = lax.fori_loop(0, _NV, cntgt, (zero16, zero16))
    gs_tot, gt_tot = _global_pair(32, gs, gt)
    ne_s = _K - gs_tot
    ne_t = _K - gt_tot

    def iround(r, carry):
        lo2s, hi2s, lo2t, hi2t = carry
        mid_s = _avg_floor(lo2s, hi2s)
        mid_t = _avg_floor(lo2t, hi2t)

        def cnt(i, acc):
            a, b = acc
            gidx = row0 + i * 16 + lane
            a = a + plsc.all_reduce_population_count(
                (us[pl.ds(i * 16, 16)] == Ts) & (gidx <= mid_s))
            b = b + plsc.all_reduce_population_count(
                (ut[pl.ds(i * 16, 16)] == Tt) & (gidx <= mid_t))
            return (a, b)

        cs, ct = lax.fori_loop(0, _NV, cnt, (zero16, zero16))
        cs_tot, ct_tot = _global_pair(33 + r, cs, ct)

        go_s = lo2s < hi2s
        take_s = cs_tot >= ne_s
        hi2s = jnp.where(go_s & take_s, mid_s, hi2s)
        lo2s = jnp.where(go_s & (~take_s), mid_s + 1, lo2s)
        go_t = lo2t < hi2t
        take_t = ct_tot >= ne_t
        hi2t = jnp.where(go_t & take_t, mid_t, hi2t)
        lo2t = jnp.where(go_t & (~take_t), mid_t + 1, lo2t)
        return (lo2s, hi2s, lo2t, hi2t)

    Ps, _, Pt, _ = lax.fori_loop(
        0, 14, iround,
        (jnp.int32(0), jnp.int32(_N - 1), jnp.int32(0), jnp.int32(_N - 1)))

    def comp(i, n):
        iv = c * (_BROWS // 16) + i
        uv = us[pl.ds(iv * 16, 16)]
        tv = ut[pl.ds(iv * 16, 16)]
        gidx = row0 + iv * 16 + lane
        keep = ((uv > Ts) | ((uv == Ts) & (gidx <= Ps)) |
                (tv > Tt) | ((tv == Tt) & (gidx <= Pt)))
        ki = keep.astype(jnp.int32)
        incl = plsc.cumsum(ki)
        pos = n + incl - ki
        plsc.store_scatter(idxk, [pos >> 4, pos & 15], gidx, mask=keep)
        plsc.store_scatter(idxv, [pos >> 4, pos & 15], _N + gidx, mask=keep)
        return n + plsc.all_reduce_population_count(keep)

    n = lax.fori_loop(0, _GROUPS, comp, zero16)
    ngroups = (_lane0(n) + 15) >> 4

    fk = _lane0(idxk[0, :])

    def padfix(g, _):
        m = (g * 16 + lane) >= n
        idxk[g, :] = jnp.where(m, fk, idxk[g, :])
        idxv[g, :] = jnp.where(m, _N + fk, idxv[g, :])
        return 0

    lax.fori_loop(0, _GROUPS, padfix, 0)

    stk = (stagek, stagek2)
    stv = (stagev, stagev2)
    sems = (sem_g, sem_g2)

    def _gather(g, b):
        kcp = pltpu.make_async_copy(keys_hbm.at[idxk.at[g]], stk[b], sems[b])
        vcp = pltpu.make_async_copy(values_hbm.at[idxk.at[g]], stv[b], sems[b])
        kcp.start()
        vcp.start()

    def _drain(g, b):
        pltpu.make_async_copy(keys_hbm.at[idxk.at[g]], stk[b], sems[b]).wait()
        pltpu.make_async_copy(values_hbm.at[idxk.at[g]], stv[b], sems[b]).wait()

    @pl.when(0 < ngroups)
    def _():
        _gather(0, 0)

    for cp in zcopies:
        cp.wait()

    ssems = (sem_s, sem_s2)

    def _scatter(g, b):
        kcp = pltpu.make_async_copy(stk[b], out_hbm.at[idxk.at[g]], ssems[b])
        vcp = pltpu.make_async_copy(stv[b], out_hbm.at[idxv.at[g]], ssems[b])
        kcp.start()
        vcp.start()

    def _sdrain(g, b):
        pltpu.make_async_copy(stk[b], out_hbm.at[idxk.at[g]], ssems[b]).wait()
        pltpu.make_async_copy(stv[b], out_hbm.at[idxv.at[g]], ssems[b]).wait()

    for g in range(_GROUPS):
        b = g & 1

        @pl.when(g < ngroups)
        def _(g=g, b=b):
            _drain(g, b)
            if g + 1 < _GROUPS:
                @pl.when(g + 1 < ngroups)
                def _():
                    if g >= 1:
                        _sdrain(g - 1, 1 - b)
                    _gather(g + 1, 1 - b)
            _scatter(g, b)

    for g in range(_GROUPS):
        b = g & 1

        @pl.when((g < ngroups) & (g + 2 >= ngroups))
        def _(g=g, b=b):
            _sdrain(g, b)


@jax.jit
def kernel(keys, values, attention_scores, timestamps):
    mesh = plsc.VectorSubcoreMesh(
        core_axis_name="c", subcore_axis_name="s", num_cores=2)
    run = functools.partial(
        pl.kernel,
        out_type=jax.ShapeDtypeStruct((2 * _N, _H), jnp.float32),
        mesh=mesh,
        compiler_params=pltpu.CompilerParams(needs_layout_passes=False),
        scratch_types=[
            pltpu.VMEM((_PER,), jnp.float32),
            pltpu.VMEM((_PER,), jnp.float32),
            pltpu.VMEM((_PER,), jnp.int32),
            pltpu.VMEM((_PER,), jnp.int32),
            pltpu.VMEM((_ZROWS, _H), jnp.float32),
            pltpu.VMEM((_GROUPS, 16), jnp.int32),
            pltpu.VMEM((_GROUPS, 16), jnp.int32),
            pltpu.VMEM((16, _H), jnp.float32),
            pltpu.VMEM((16, _H), jnp.float32),
            pltpu.VMEM((16, _H), jnp.float32),
            pltpu.VMEM((16, _H), jnp.float32),
            pltpu.SMEM((2 * _RPAD,), jnp.int32),
            pltpu.SemaphoreType.DMA,
            pltpu.SemaphoreType.DMA,
            pltpu.SemaphoreType.DMA,
            pltpu.SemaphoreType.DMA,
            pltpu.SemaphoreType.DMA,
        ],
    )(_sc_body)
    flat = run(keys, values, attention_scores, timestamps)
    return flat.reshape(2, _N, _H)
